# baseline (speedup 1.0000x reference)
import functools

import jax
import jax.numpy as jnp
from jax import lax
from jax.experimental import pallas as pl
from jax.experimental.pallas import tpu as pltpu
from jax.experimental.pallas import tpu_sc as plsc

_T = 2048
_D = 1024
_E = 8
_F = 512
_SF = 2
_LIMIT = 7.0
_RSF = 2.5

_BG = 128
_BC = 256
_PMAX = _T * 2 // _BG * _BG + _E * _BG + _BC
_PMAX = ((_PMAX + _BC - 1) // _BC) * _BC
_CHUNK = 256
_NCH = 2 * _T // _CHUNK
_FC = 128
_NFC = _F // _FC
_SFC = _F * _SF // _NFC
_NW = 32
_TPW = _T // _NW
_HT = _TPW // 2
_BT = 256


def _router_kernel(x_ref, gw_ref, sg_ref, su_ref, sd_ref,
                   s_ref, pos_ref, w_ref, info_ref, oh_ref, cum_ref):
    t = pl.program_id(0)
    row = pl.ds(t * _BT, _BT)
    xb = x_ref[row, :].astype(jnp.bfloat16)
    sgb = sg_ref[...].astype(jnp.bfloat16)
    sub = su_ref[...].astype(jnp.bfloat16)
    sdb = sd_ref[...].astype(jnp.bfloat16)
    dn = (((1,), (1,)), ((), ()))
    a = lax.dot_general(xb, sgb, dn, preferred_element_type=jnp.float32)
    b = lax.dot_general(xb, sub, dn, preferred_element_type=jnp.float32)
    hs = (a * jax.nn.sigmoid(a) * b).astype(jnp.bfloat16)
    s_ref[...] = lax.dot_general(hs, sdb, dn, preferred_element_type=jnp.float32)

    @pl.when(t == 0)
    def _router():
        _router_body(x_ref, gw_ref, pos_ref, w_ref, info_ref, oh_ref, cum_ref)


def _router_body(x_ref, gw_ref, pos_ref, w_ref, info_ref, oh_ref, cum_ref):
    x = x_ref[...]
    logits = jnp.dot(x, gw_ref[...].T, preferred_element_type=jnp.float32)
    scores = jnp.sqrt(jax.nn.softplus(logits))
    col = lax.broadcasted_iota(jnp.int32, scores.shape, 1)
    m1 = jnp.max(scores, axis=1, keepdims=True)
    i1 = jnp.min(jnp.where(scores == m1, col, _E), axis=1, keepdims=True)
    masked = jnp.where(col == i1, -jnp.inf, scores)
    m2 = jnp.max(masked, axis=1, keepdims=True)
    i2 = jnp.min(jnp.where(masked == m2, col, _E), axis=1, keepdims=True)
    s = m1 + m2
    col2 = lax.broadcasted_iota(jnp.int32, (_T, 2), 1)
    w_ref[...] = jnp.where(col2 == 0, m1, m2) * (_RSF / s)

    oh_ref[0:_T, :] = (col == i1).astype(jnp.float32)
    oh_ref[_T:2 * _T, :] = (col == i2).astype(jnp.float32)

    sel_r = lax.broadcasted_iota(jnp.int32, (_NCH, 2 * _T), 0)
    sel_c = lax.broadcasted_iota(jnp.int32, (_NCH, 2 * _T), 1)
    sel = (sel_r == sel_c // _CHUNK).astype(jnp.float32)
    s16 = jnp.dot(sel, oh_ref[...], preferred_element_type=jnp.float32)
    t_r = lax.broadcasted_iota(jnp.int32, (_NCH, _NCH), 0)
    t_c = lax.broadcasted_iota(jnp.int32, (_NCH, _NCH), 1)
    tri16 = (t_r > t_c).astype(jnp.float32)
    off16 = jnp.dot(tri16, s16, preferred_element_type=jnp.float32)

    r_io = lax.broadcasted_iota(jnp.int32, (_CHUNK, _CHUNK), 0)
    c_io = lax.broadcasted_iota(jnp.int32, (_CHUNK, _CHUNK), 1)
    tri = (r_io > c_io).astype(jnp.float32)
    for i in range(_NCH):
        sl = pl.ds(i * _CHUNK, _CHUNK)
        a = oh_ref[sl, :]
        cum_ref[sl, :] = (jnp.dot(tri, a, preferred_element_type=jnp.float32)
                          + off16[i:i + 1, :])

    tot = jnp.sum(s16, axis=0, keepdims=True)
    padded = float(_BG) * jnp.floor((tot + float(_BG - 1)) / float(_BG))
    e_r = lax.broadcasted_iota(jnp.int32, (_E, _E), 0)
    e_c = lax.broadcasted_iota(jnp.int32, (_E, _E), 1)
    m8 = (e_r < e_c).astype(jnp.float32)
    base = jnp.dot(padded, m8, preferred_element_type=jnp.float32)

    p0 = jnp.sum(jnp.where(col == i1, base + cum_ref[0:_T, :], 0.0),
                 axis=1, keepdims=True)
    p1 = jnp.sum(jnp.where(col == i2, base + cum_ref[_T:2 * _T, :], 0.0),
                 axis=1, keepdims=True)
    pos_ref[...] = jnp.where(col2 == 0, p0, p1).astype(jnp.int32)

    lane32 = lax.broadcasted_iota(jnp.int32, (1, 32), 1)
    info = jnp.zeros((1, 32), jnp.float32)
    for e in range(_E):
        info = info + jnp.where(lane32 == e, base[0, e] / float(_BG), 0.0)
        info = info + jnp.where(
            lane32 == 8 + e,
            jnp.floor((padded[0, e] / float(_BG) + 1.0) / 2.0), 0.0)
    info_ref[...] = info.astype(jnp.int32)


def _run_router(x, gate_w, shared_gate, shared_up, shared_down):
    nt = _T // _BT
    return pl.pallas_call(
        _router_kernel,
        grid=(nt,),
        in_specs=[
            pl.BlockSpec((_T, _D), lambda t: (0, 0)),
            pl.BlockSpec((_E, _D), lambda t: (0, 0)),
            pl.BlockSpec((_F * _SF, _D), lambda t: (0, 0)),
            pl.BlockSpec((_F * _SF, _D), lambda t: (0, 0)),
            pl.BlockSpec((_D, _F * _SF), lambda t: (0, 0)),
        ],
        out_specs=[
            pl.BlockSpec((_BT, _D), lambda t: (t, 0)),
            pl.BlockSpec((_T, 2), lambda t: (0, 0)),
            pl.BlockSpec((_T, 2), lambda t: (0, 0)),
            pl.BlockSpec((1, 32), lambda t: (0, 0)),
        ],
        out_shape=[
            jax.ShapeDtypeStruct((_T, _D), jnp.float32),
            jax.ShapeDtypeStruct((_T, 2), jnp.int32),
            jax.ShapeDtypeStruct((_T, 2), jnp.float32),
            jax.ShapeDtypeStruct((1, 32), jnp.int32),
        ],
        scratch_shapes=[
            pltpu.VMEM((2 * _T, _E), jnp.float32),
            pltpu.VMEM((2 * _T, _E), jnp.float32),
        ],
    )(x, gate_w, shared_gate, shared_up, shared_down)


def _make_scatter():
    mesh = plsc.VectorSubcoreMesh(core_axis_name="c", subcore_axis_name="s")

    @functools.partial(
        pl.kernel, mesh=mesh,
        out_type=jax.ShapeDtypeStruct((_PMAX, _D), jnp.float32),
        scratch_types=[
            pltpu.VMEM((_TPW, _D), jnp.float32),
            pltpu.VMEM((_TPW,), jnp.int32),
            pltpu.VMEM((_TPW,), jnp.int32),
            pltpu.SemaphoreType.DMA,
        ],
    )
    def scatter_k(x_hbm, pos0_hbm, pos1_hbm, xs_hbm, xbuf, i0, i1, sem):
        wid = lax.axis_index("s") * 2 + lax.axis_index("c")
        base = wid * _TPW
        pltpu.sync_copy(x_hbm.at[pl.ds(base, _TPW)], xbuf)
        pltpu.sync_copy(pos0_hbm.at[pl.ds(base, _TPW)], i0)
        pltpu.sync_copy(pos1_hbm.at[pl.ds(base, _TPW)], i1)
        c0 = pltpu.async_copy(xbuf, xs_hbm.at[i0], sem)
        c1 = pltpu.async_copy(xbuf, xs_hbm.at[i1], sem)
        c0.wait()
        c1.wait()

    return scatter_k


def _ffn_kernel(info_ref, xs_ref, wg_ref, wu_ref, wd_ref, y_ref):
    e = pl.program_id(0)
    dn = (((1,), (1,)), ((), ()))
    start = info_ref[e]
    nch = info_ref[8 + e]
    wg16 = wg_ref[0].astype(jnp.bfloat16)
    wu16 = wu_ref[0].astype(jnp.bfloat16)
    wd16 = wd_ref[0].astype(jnp.bfloat16)

    def chunk(c, carry):
        row0 = pl.multiple_of(start * _BG, _BG) + c * _BC
        rows = pl.ds(row0, _BC)
        xb = xs_ref[rows, :].astype(jnp.bfloat16)
        g = lax.dot_general(xb, wg16, dn, preferred_element_type=jnp.float32)
        u = lax.dot_general(xb, wu16, dn, preferred_element_type=jnp.float32)
        g = jnp.minimum(g, _LIMIT)
        u = jnp.clip(u, -_LIMIT, _LIMIT)
        h = ((g * jax.nn.sigmoid(g)) * u).astype(jnp.bfloat16)
        y_ref[rows, :] = lax.dot_general(h, wd16, dn,
                                         preferred_element_type=jnp.float32)
        return carry

    lax.fori_loop(0, nch, chunk, 0)


def _run_ffn(xs, w_gate, w_up, w_down, info):
    grid_spec = pltpu.PrefetchScalarGridSpec(
        num_scalar_prefetch=1,
        grid=(_E,),
        in_specs=[
            pl.BlockSpec((_PMAX, _D), lambda e, info: (0, 0)),
            pl.BlockSpec((1, _F, _D), lambda e, info: (e, 0, 0)),
            pl.BlockSpec((1, _F, _D), lambda e, info: (e, 0, 0)),
            pl.BlockSpec((1, _D, _F), lambda e, info: (e, 0, 0)),
        ],
        out_specs=pl.BlockSpec((_PMAX, _D), lambda e, info: (0, 0)),
    )
    return pl.pallas_call(
        _ffn_kernel,
        grid_spec=grid_spec,
        out_shape=jax.ShapeDtypeStruct((_PMAX, _D), jnp.float32),
        compiler_params=pltpu.CompilerParams(
            vmem_limit_bytes=62 * 1024 * 1024),
    )(info, xs, w_gate, w_up, w_down)


def _make_gather():
    mesh = plsc.VectorSubcoreMesh(core_axis_name="c", subcore_axis_name="s")

    @functools.partial(
        pl.kernel, mesh=mesh,
        out_type=[
            jax.ShapeDtypeStruct((_T, _D), jnp.float32),
            jax.ShapeDtypeStruct((_T, _D), jnp.float32),
        ],
        scratch_types=[
            pltpu.VMEM((_TPW, _D), jnp.float32),
            pltpu.VMEM((_TPW,), jnp.int32),
            pltpu.SemaphoreType.DMA,
        ],
    )
    def gather_k(y_hbm, pos0_hbm, pos1_hbm, y0_hbm, y1_hbm, buf, idx, sem):
        wid = lax.axis_index("s") * 2 + lax.axis_index("c")
        base = wid * _TPW
        pltpu.sync_copy(pos0_hbm.at[pl.ds(base, _TPW)], idx)
        pltpu.async_copy(y_hbm.at[idx], buf, sem).wait()
        pltpu.sync_copy(buf, y0_hbm.at[pl.ds(base, _TPW)])
        pltpu.sync_copy(pos1_hbm.at[pl.ds(base, _TPW)], idx)
        pltpu.async_copy(y_hbm.at[idx], buf, sem).wait()
        pltpu.sync_copy(buf, y1_hbm.at[pl.ds(base, _TPW)])

    return gather_k


def _combine_kernel(s_ref, y0_ref, y1_ref, w_ref, out_ref):
    w = w_ref[...]
    out_ref[...] = (s_ref[...] + w[:, 0:1] * y0_ref[...]
                    + w[:, 1:2] * y1_ref[...])


def _run_combine(shared, y0, y1, w2):
    nt = _T // _BT
    return pl.pallas_call(
        _combine_kernel,
        grid=(nt,),
        in_specs=[
            pl.BlockSpec((_BT, _D), lambda t: (t, 0)),
            pl.BlockSpec((_BT, _D), lambda t: (t, 0)),
            pl.BlockSpec((_BT, _D), lambda t: (t, 0)),
            pl.BlockSpec((_BT, 2), lambda t: (t, 0)),
        ],
        out_specs=pl.BlockSpec((_BT, _D), lambda t: (t, 0)),
        out_shape=jax.ShapeDtypeStruct((_T, _D), jnp.float32),
    )(shared, y0, y1, w2)


def kernel(hidden_states, gate_w, w_gate, w_up, w_down,
           shared_gate, shared_up, shared_down):
    org_shape = hidden_states.shape
    x = hidden_states.reshape(-1, org_shape[-1])

    shared, pos2, w2, info = _run_router(x, gate_w, shared_gate,
                                         shared_up, shared_down)
    pos0 = pos2[:, 0]
    pos1 = pos2[:, 1]

    xs = _make_scatter()(x, pos0, pos1)
    y = _run_ffn(xs, w_gate, w_up, w_down, info.reshape(32))
    y0, y1 = _make_gather()(y, pos0, pos1)
    out = _run_combine(shared, y0, y1, w2)
    return out.reshape(org_shape)
